# Initial kernel scaffold; baseline (speedup 1.0000x reference)
import jax
import jax.numpy as jnp
from jax.experimental import pallas as pl


def kernel(x, edge_index, W_src0, W_dst0, att_src0, att_dst0, bias0, bn_gamma, bn_beta, bn_mean, bn_var, W1, att_src1, att_dst1, bias1):
    raise NotImplementedError("write your pallas kernel here")


# speedup vs baseline: 12.8625x; 12.8625x over previous
import functools

import jax
import jax.numpy as jnp
from jax import lax
from jax.experimental import pallas as pl
from jax.experimental.pallas import tpu as pltpu
from jax.experimental.pallas import tpu_sc as plsc

_N = 10000
_E = 160000
_D = 256
_HID = 512
_HEADS = 4
_C = 128
_S = 64
_NP = 10240
_TILES = 16
_CPB = 79
_CH = _CPB * 128
_EP = _TILES * _CH
_PAD_DST = _N + 16
_RPT = _NP // _TILES
_TN = 512


def _proj0_body(x_ref, ws_ref, wd_ref, ats_ref, atd_ref,
                hs_ref, as_ref, ad_ref):
    h = pl.program_id(0)
    i = pl.program_id(1)
    xb = x_ref[...]
    hs = jnp.dot(xb, ws_ref[...], preferred_element_type=jnp.float32)
    hd = jnp.dot(xb, wd_ref[...], preferred_element_type=jnp.float32)
    hs_ref[0] = hs[:, :_S]
    hs_ref[1] = hs[:, _S:]
    sl = pl.ds(i * _TN, _TN)
    as_ref[h, sl] = jnp.sum(hs * ats_ref[h][None, :], axis=1)
    ad_ref[h, sl] = jnp.sum(hd * atd_ref[h][None, :], axis=1)


def _proj0(xp, w_src, w_dst, att_src, att_dst):
    grid = (_HEADS, _NP // _TN)
    return pl.pallas_call(
        _proj0_body,
        grid=grid,
        in_specs=[
            pl.BlockSpec((_TN, _D), lambda h, i: (i, 0)),
            pl.BlockSpec((_D, _C), lambda h, i: (0, h)),
            pl.BlockSpec((_D, _C), lambda h, i: (0, h)),
            pl.BlockSpec((_HEADS, _C), lambda h, i: (0, 0)),
            pl.BlockSpec((_HEADS, _C), lambda h, i: (0, 0)),
        ],
        out_specs=[
            pl.BlockSpec((2, _TN, _S), lambda h, i: (h, i, 0)),
            pl.BlockSpec((_HEADS, _NP), lambda h, i: (0, 0)),
            pl.BlockSpec((_HEADS, _NP), lambda h, i: (0, 0)),
        ],
        out_shape=[
            jax.ShapeDtypeStruct((2 * _HEADS, _NP, _S), jnp.float32),
            jax.ShapeDtypeStruct((_HEADS, _NP), jnp.float32),
            jax.ShapeDtypeStruct((_HEADS, _NP), jnp.float32),
        ],
    )(xp, w_src, w_dst, att_src, att_dst)


def _mlp_body(v_ref, b0_ref, ga_ref, be_ref, mu_ref, va_ref, w1_ref,
              s1_ref, d1_ref, h1_ref, a1s_ref, a1d_ref, acc_ref):
    i = pl.program_id(0)
    j = pl.program_id(1)
    g = pl.program_id(2)

    def act(v, k):
        scale = ga_ref[k][None, :] * lax.rsqrt(va_ref[k][None, :] + 1e-5)
        a = (v + b0_ref[k][None, :] - mu_ref[k][None, :]) * scale \
            + be_ref[k][None, :]
        return jnp.where(a > 0, a, jnp.exp(a) - 1.0)

    aa = act(v_ref[0], 2 * g)
    ab = act(v_ref[1], 2 * g + 1)
    part = jnp.dot(aa, w1_ref[2 * g, :, j, :],
                   preferred_element_type=jnp.float32)
    part += jnp.dot(ab, w1_ref[2 * g + 1, :, j, :],
                    preferred_element_type=jnp.float32)

    @pl.when(g == 0)
    def _():
        acc_ref[...] = part

    @pl.when(g > 0)
    def _():
        acc_ref[...] += part

    @pl.when(g == _HEADS - 1)
    def _():
        h1 = acc_ref[...]
        h1_ref[0] = h1[:, :_S]
        h1_ref[1] = h1[:, _S:]
        ps = jnp.sum(h1 * s1_ref[j][None, :], axis=1)
        pd = jnp.sum(h1 * d1_ref[j][None, :], axis=1)
        sl = pl.ds(i * _TN, _TN)

        @pl.when(j == 0)
        def _():
            a1s_ref[0, sl] = ps
            a1d_ref[0, sl] = pd

        @pl.when(j == 1)
        def _():
            a1s_ref[0, sl] += ps
            a1d_ref[0, sl] += pd


def _mlp(out0g, b0r, gar, ber, mur, var_, w1r, s1r, d1r):
    grid = (_NP // _TN, 2, _HEADS)
    full2 = pl.BlockSpec((2 * _HEADS, _S), lambda i, j, g: (0, 0))
    return pl.pallas_call(
        _mlp_body,
        grid=grid,
        in_specs=[
            pl.BlockSpec((2, _TN, _S), lambda i, j, g: (g, i, 0)),
            full2, full2, full2, full2, full2,
            pl.BlockSpec((2 * _HEADS, _S, 2, _C),
                         lambda i, j, g: (0, 0, 0, 0)),
            pl.BlockSpec((2, _C), lambda i, j, g: (0, 0)),
            pl.BlockSpec((2, _C), lambda i, j, g: (0, 0)),
        ],
        out_specs=[
            pl.BlockSpec((2, _TN, _S), lambda i, j, g: (j, i, 0)),
            pl.BlockSpec((1, _NP), lambda i, j, g: (0, 0)),
            pl.BlockSpec((1, _NP), lambda i, j, g: (0, 0)),
        ],
        out_shape=[
            jax.ShapeDtypeStruct((4, _NP, _S), jnp.float32),
            jax.ShapeDtypeStruct((1, _NP), jnp.float32),
            jax.ShapeDtypeStruct((1, _NP), jnp.float32),
        ],
        scratch_shapes=[pltpu.VMEM((_TN, _C), jnp.float32)],
    )(out0g, b0r, gar, ber, mur, var_, w1r, s1r, d1r)


def _zero_rows(rows_v):
    z16 = jnp.zeros((16,), jnp.float32)

    def zrow(r, c):
        for q in range(_S // 16):
            rows_v[r, pl.ds(q * 16, 16)] = z16
        return c

    lax.fori_loop(0, 128, zrow, 0)


def _sc_half(t, src_v, dst_v, w_v, rows_v, acc_sh, sem, hs_h, out_h):
    base = t * _RPT
    _zero_rows(rows_v)
    for kk in range(_RPT // 128):
        pltpu.sync_copy(rows_v, acc_sh.at[pl.ds(base + kk * 128, 128)])
    plsc.subcore_barrier()

    def pass_b(j, carry):
        pltpu.async_copy(hs_h.at[src_v.at[j]], rows_v, sem).wait()
        js = jnp.zeros((16,), jnp.int32) + j

        def scale_row(r, c2):
            av = plsc.load_gather(w_v, [js, jnp.zeros((16,), jnp.int32) + r])
            for q in range(_S // 16):
                sl = pl.ds(q * 16, 16)
                rows_v[r, sl] = rows_v[r, sl] * av
            return c2

        lax.fori_loop(0, 128, scale_row, 0)
        pltpu.sync_copy(rows_v, acc_sh.at[dst_v.at[j]], add=True)
        return carry

    lax.fori_loop(0, _CPB, pass_b, 0)
    plsc.subcore_barrier()
    pltpu.sync_copy(acc_sh.at[pl.ds(base, _RPT)], out_h.at[pl.ds(base, _RPT)])
    plsc.subcore_barrier()


def _sc_group(t, src_v, dst_v, w_v, asv, adv, denv, zrow_v, rows_v,
              den_sh, acc_sh, sem,
              as_h, ad_h, hsa_h, hsb_h, outa_h, outb_h):
    base = t * _RPT
    for kk in range(_RPT // 128):
        pltpu.sync_copy(zrow_v, den_sh.at[pl.ds(base + kk * 128, 128)])
    pltpu.sync_copy(as_h, asv)
    pltpu.sync_copy(ad_h, adv)
    plsc.subcore_barrier()

    def pass_a(j, carry):
        for q in range(8):
            sl = pl.ds(q * 16, 16)
            sv = src_v[j, sl]
            dv = dst_v[j, sl]
            e = plsc.load_gather(asv, [sv]) + plsc.load_gather(adv, [dv])
            e = jnp.maximum(e, 0.2 * e)
            w_v[j, sl] = jnp.exp(e)
        pltpu.sync_copy(w_v.at[j], den_sh.at[dst_v.at[j]], add=True)
        return carry

    lax.fori_loop(0, _CPB, pass_a, 0)
    plsc.subcore_barrier()
    pltpu.sync_copy(den_sh, denv)

    def to_alpha(j, carry):
        for q in range(8):
            sl = pl.ds(q * 16, 16)
            dn = plsc.load_gather(denv, [dst_v[j, sl]])
            w_v[j, sl] = w_v[j, sl] / (dn + 1e-16)
        return carry

    lax.fori_loop(0, _CPB, to_alpha, 0)

    _sc_half(t, src_v, dst_v, w_v, rows_v, acc_sh, sem, hsa_h, outa_h)
    _sc_half(t, src_v, dst_v, w_v, rows_v, acc_sh, sem, hsb_h, outb_h)


def _sc_scratch():
    return [
        pltpu.VMEM((_CPB, 128), jnp.int32),
        pltpu.VMEM((_CPB, 128), jnp.int32),
        pltpu.VMEM((_CPB, 128), jnp.float32),
        pltpu.VMEM((_NP,), jnp.float32),
        pltpu.VMEM((_NP,), jnp.float32),
        pltpu.VMEM((_NP,), jnp.float32),
        pltpu.VMEM((128,), jnp.float32),
        pltpu.VMEM((128, _S), jnp.float32),
        pltpu.VMEM_SHARED((_NP,), jnp.float32),
        pltpu.VMEM_SHARED((_NP, _S), jnp.float32),
        pltpu.SemaphoreType.DMA,
    ]


def _sc_prologue(src_h, dst_h, t, src_v, dst_v, zrow_v):
    pltpu.sync_copy(src_h.at[t], src_v)
    pltpu.sync_copy(dst_h.at[t], dst_v)
    z16 = jnp.zeros((16,), jnp.float32)
    for q in range(8):
        zrow_v[pl.ds(q * 16, 16)] = z16


_SC_MESH = plsc.VectorSubcoreMesh(core_axis_name="c", subcore_axis_name="s")


@functools.partial(
    pl.kernel,
    out_type=jax.ShapeDtypeStruct((2 * _HEADS, _NP, _S), jnp.float32),
    mesh=_SC_MESH,
    compiler_params=pltpu.CompilerParams(
        needs_layout_passes=False, use_tc_tiling_on_sc=False),
    scratch_types=_sc_scratch(),
)
def _gat0(src_h, dst_h, as_h, ad_h, hs_h, out_h,
          src_v, dst_v, w_v, asv, adv, denv, zrow_v, rows_v,
          den_sh, acc_sh, sem):
    c = lax.axis_index("c")
    t = lax.axis_index("s")
    _sc_prologue(src_h, dst_h, t, src_v, dst_v, zrow_v)
    args = (src_v, dst_v, w_v, asv, adv, denv, zrow_v, rows_v,
            den_sh, acc_sh, sem)

    @pl.when(c == 0)
    def _():
        for g in (0, 1):
            _sc_group(t, *args, as_h.at[g], ad_h.at[g],
                      hs_h.at[2 * g], hs_h.at[2 * g + 1],
                      out_h.at[2 * g], out_h.at[2 * g + 1])

    @pl.when(c == 1)
    def _():
        for g in (2, 3):
            _sc_group(t, *args, as_h.at[g], ad_h.at[g],
                      hs_h.at[2 * g], hs_h.at[2 * g + 1],
                      out_h.at[2 * g], out_h.at[2 * g + 1])


@functools.partial(
    pl.kernel,
    out_type=jax.ShapeDtypeStruct((4, _NP, _S), jnp.float32),
    mesh=_SC_MESH,
    compiler_params=pltpu.CompilerParams(
        needs_layout_passes=False, use_tc_tiling_on_sc=False),
    scratch_types=_sc_scratch(),
)
def _gat1(src_h, dst_h, as_h, ad_h, h1_h, out_h,
          src_v, dst_v, w_v, asv, adv, denv, zrow_v, rows_v,
          den_sh, acc_sh, sem):
    c = lax.axis_index("c")
    t = lax.axis_index("s")
    _sc_prologue(src_h, dst_h, t, src_v, dst_v, zrow_v)
    args = (src_v, dst_v, w_v, asv, adv, denv, zrow_v, rows_v,
            den_sh, acc_sh, sem)

    @pl.when(c == 0)
    def _():
        _sc_group(t, *args, as_h.at[0], ad_h.at[0],
                  h1_h.at[0], h1_h.at[1], out_h.at[0], out_h.at[1])

    @pl.when(c == 1)
    def _():
        _sc_group(t, *args, as_h.at[0], ad_h.at[0],
                  h1_h.at[2], h1_h.at[3], out_h.at[2], out_h.at[3])


def kernel(x, edge_index, W_src0, W_dst0, att_src0, att_dst0, bias0,
           bn_gamma, bn_beta, bn_mean, bn_var, W1, att_src1, att_dst1,
           bias1):
    xp = jnp.pad(x, ((0, _NP - _N), (0, 0)))
    src = edge_index[0]
    dst = edge_index[1]
    srcp = jnp.concatenate(
        [src, jnp.zeros((_EP - _E,), jnp.int32)]).reshape(_TILES, _CPB, 128)
    dstp = jnp.concatenate(
        [dst, jnp.full((_EP - _E,), _PAD_DST, jnp.int32)]
    ).reshape(_TILES, _CPB, 128)

    hs, a_s, a_d = _proj0(xp, W_src0, W_dst0, att_src0, att_dst0)
    out0g = _gat0(srcp, dstp, a_s, a_d, hs)

    h1g, a1s, a1d = _mlp(
        out0g,
        bias0.reshape(2 * _HEADS, _S),
        bn_gamma.reshape(2 * _HEADS, _S),
        bn_beta.reshape(2 * _HEADS, _S),
        bn_mean.reshape(2 * _HEADS, _S),
        bn_var.reshape(2 * _HEADS, _S),
        W1.reshape(2 * _HEADS, _S, 2, _C),
        att_src1.reshape(2, _C),
        att_dst1.reshape(2, _C),
    )
    out1g = _gat1(srcp, dstp, a1s, a1d, h1g)
    out = out1g.transpose(1, 0, 2).reshape(_NP, 4 * _S)[:_N] + bias1
    return out


# speedup vs baseline: 16.1161x; 1.2530x over previous
import functools

import jax
import jax.numpy as jnp
from jax import lax
from jax.experimental import pallas as pl
from jax.experimental.pallas import tpu as pltpu
from jax.experimental.pallas import tpu_sc as plsc

_N = 10000
_E = 160000
_D = 256
_HID = 512
_HEADS = 4
_C = 128
_S = 64
_NP = 10240
_TILES = 16
_CPB = 79
_CH = _CPB * 128
_EP = _TILES * _CH
_PAD_DST = _N + 16
_RPT = _NP // _TILES
_TN = 512


def _proj0_body(x_ref, ws_ref, wd_ref, ats_ref, atd_ref,
                hs_ref, as_ref, ad_ref):
    h = pl.program_id(0)
    i = pl.program_id(1)
    xb = x_ref[...]
    hs = jnp.dot(xb, ws_ref[...], preferred_element_type=jnp.float32)
    hd = jnp.dot(xb, wd_ref[...], preferred_element_type=jnp.float32)
    hs_ref[0] = hs[:, :_S]
    hs_ref[1] = hs[:, _S:]
    sl = pl.ds(i * _TN, _TN)
    as_ref[h, sl] = jnp.sum(hs * ats_ref[h][None, :], axis=1)
    ad_ref[h, sl] = jnp.sum(hd * atd_ref[h][None, :], axis=1)


def _proj0(xp, w_src, w_dst, att_src, att_dst):
    grid = (_HEADS, _NP // _TN)
    return pl.pallas_call(
        _proj0_body,
        grid=grid,
        in_specs=[
            pl.BlockSpec((_TN, _D), lambda h, i: (i, 0)),
            pl.BlockSpec((_D, _C), lambda h, i: (0, h)),
            pl.BlockSpec((_D, _C), lambda h, i: (0, h)),
            pl.BlockSpec((_HEADS, _C), lambda h, i: (0, 0)),
            pl.BlockSpec((_HEADS, _C), lambda h, i: (0, 0)),
        ],
        out_specs=[
            pl.BlockSpec((2, _TN, _S), lambda h, i: (h, i, 0)),
            pl.BlockSpec((_HEADS, _NP), lambda h, i: (0, 0)),
            pl.BlockSpec((_HEADS, _NP), lambda h, i: (0, 0)),
        ],
        out_shape=[
            jax.ShapeDtypeStruct((2 * _HEADS, _NP, _S), jnp.float32),
            jax.ShapeDtypeStruct((_HEADS, _NP), jnp.float32),
            jax.ShapeDtypeStruct((_HEADS, _NP), jnp.float32),
        ],
    )(xp, w_src, w_dst, att_src, att_dst)


def _mlp_body(v_ref, b0_ref, ga_ref, be_ref, mu_ref, va_ref, w1_ref,
              s1_ref, d1_ref, h1_ref, a1s_ref, a1d_ref, acc_ref):
    i = pl.program_id(0)
    j = pl.program_id(1)
    g = pl.program_id(2)

    def act(v, k):
        scale = ga_ref[k][None, :] * lax.rsqrt(va_ref[k][None, :] + 1e-5)
        a = (v + b0_ref[k][None, :] - mu_ref[k][None, :]) * scale \
            + be_ref[k][None, :]
        return jnp.where(a > 0, a, jnp.exp(a) - 1.0)

    aa = act(v_ref[0], 2 * g)
    ab = act(v_ref[1], 2 * g + 1)
    part = jnp.dot(aa, w1_ref[2 * g, :, j, :],
                   preferred_element_type=jnp.float32)
    part += jnp.dot(ab, w1_ref[2 * g + 1, :, j, :],
                    preferred_element_type=jnp.float32)

    @pl.when(g == 0)
    def _():
        acc_ref[...] = part

    @pl.when(g > 0)
    def _():
        acc_ref[...] += part

    @pl.when(g == _HEADS - 1)
    def _():
        h1 = acc_ref[...]
        h1_ref[0] = h1[:, :_S]
        h1_ref[1] = h1[:, _S:]
        ps = jnp.sum(h1 * s1_ref[j][None, :], axis=1)
        pd = jnp.sum(h1 * d1_ref[j][None, :], axis=1)
        sl = pl.ds(i * _TN, _TN)

        @pl.when(j == 0)
        def _():
            a1s_ref[0, sl] = ps
            a1d_ref[0, sl] = pd

        @pl.when(j == 1)
        def _():
            a1s_ref[0, sl] += ps
            a1d_ref[0, sl] += pd


def _mlp(out0g, b0r, gar, ber, mur, var_, w1r, s1r, d1r):
    grid = (_NP // _TN, 2, _HEADS)
    full2 = pl.BlockSpec((2 * _HEADS, _S), lambda i, j, g: (0, 0))
    return pl.pallas_call(
        _mlp_body,
        grid=grid,
        in_specs=[
            pl.BlockSpec((2, _TN, _S), lambda i, j, g: (g, i, 0)),
            full2, full2, full2, full2, full2,
            pl.BlockSpec((2 * _HEADS, _S, 2, _C),
                         lambda i, j, g: (0, 0, 0, 0)),
            pl.BlockSpec((2, _C), lambda i, j, g: (0, 0)),
            pl.BlockSpec((2, _C), lambda i, j, g: (0, 0)),
        ],
        out_specs=[
            pl.BlockSpec((2, _TN, _S), lambda i, j, g: (j, i, 0)),
            pl.BlockSpec((1, _NP), lambda i, j, g: (0, 0)),
            pl.BlockSpec((1, _NP), lambda i, j, g: (0, 0)),
        ],
        out_shape=[
            jax.ShapeDtypeStruct((4, _NP, _S), jnp.float32),
            jax.ShapeDtypeStruct((1, _NP), jnp.float32),
            jax.ShapeDtypeStruct((1, _NP), jnp.float32),
        ],
        scratch_shapes=[pltpu.VMEM((_TN, _C), jnp.float32)],
    )(out0g, b0r, gar, ber, mur, var_, w1r, s1r, d1r)


def _zero_rows(rows_v):
    z16 = jnp.zeros((16,), jnp.float32)

    def zrow(r, c):
        for q in range(_S // 16):
            rows_v[r, pl.ds(q * 16, 16)] = z16
        return c

    lax.fori_loop(0, 128, zrow, 0)


def _sc_half(t, src_v, dst_v, w_v, rows0, rows1, acc_sh,
             semg0, semg1, sems0, sems1, hs_h, out_h):
    base = t * _RPT
    _zero_rows(rows0)
    for kk in range(_RPT // 128):
        pltpu.sync_copy(rows0, acc_sh.at[pl.ds(base + kk * 128, 128)])
    plsc.subcore_barrier()

    def scale(j, rows):
        js = jnp.zeros((16,), jnp.int32) + j

        def scale_row(r, c2):
            av = plsc.load_gather(w_v, [js, jnp.zeros((16,), jnp.int32) + r])
            for q in range(_S // 16):
                sl = pl.ds(q * 16, 16)
                rows[r, sl] = rows[r, sl] * av
            return c2

        lax.fori_loop(0, 128, scale_row, 0)

    def wait_gather(rows, sem):
        pltpu.make_async_copy(hs_h.at[src_v.at[0]], rows, sem).wait()

    def wait_scatter(rows, sem):
        pltpu.make_async_copy(rows, acc_sh.at[dst_v.at[0]], sem).wait()

    pltpu.async_copy(hs_h.at[src_v.at[0]], rows0, semg0)

    def pair(p, carry):
        j = 2 * p

        @pl.when(p > 0)
        def _():
            wait_scatter(rows1, sems1)
        pltpu.async_copy(hs_h.at[src_v.at[j + 1]], rows1, semg1)
        wait_gather(rows0, semg0)
        scale(j, rows0)
        pltpu.async_copy(rows0, acc_sh.at[dst_v.at[j]], sems0, add=True)
        wait_gather(rows1, semg1)
        scale(j + 1, rows1)
        wait_scatter(rows0, sems0)
        pltpu.async_copy(hs_h.at[src_v.at[j + 2]], rows0, semg0)
        pltpu.async_copy(rows1, acc_sh.at[dst_v.at[j + 1]], sems1, add=True)
        return carry

    lax.fori_loop(0, (_CPB - 1) // 2, pair, 0)
    wait_scatter(rows1, sems1)
    wait_gather(rows0, semg0)
    scale(_CPB - 1, rows0)
    pltpu.async_copy(rows0, acc_sh.at[dst_v.at[_CPB - 1]], sems0, add=True)
    wait_scatter(rows0, sems0)
    plsc.subcore_barrier()
    pltpu.sync_copy(acc_sh.at[pl.ds(base, _RPT)], out_h.at[pl.ds(base, _RPT)])
    plsc.subcore_barrier()


def _sc_group(t, src_v, dst_v, w_v, asv, adv, denv, zrow_v, rows0, rows1,
              den_sh, acc_sh, semg0, semg1, sems0, sems1,
              as_h, ad_h, hsa_h, hsb_h, outa_h, outb_h):
    base = t * _RPT
    for kk in range(_RPT // 128):
        pltpu.sync_copy(zrow_v, den_sh.at[pl.ds(base + kk * 128, 128)])
    pltpu.sync_copy(as_h, asv)
    pltpu.sync_copy(ad_h, adv)
    plsc.subcore_barrier()

    def pass_a(j, carry):
        for q in range(8):
            sl = pl.ds(q * 16, 16)
            sv = src_v[j, sl]
            dv = dst_v[j, sl]
            e = plsc.load_gather(asv, [sv]) + plsc.load_gather(adv, [dv])
            e = jnp.maximum(e, 0.2 * e)
            w_v[j, sl] = jnp.exp(e)
        pltpu.sync_copy(w_v.at[j], den_sh.at[dst_v.at[j]], add=True)
        return carry

    lax.fori_loop(0, _CPB, pass_a, 0)
    plsc.subcore_barrier()
    pltpu.sync_copy(den_sh, denv)

    def to_alpha(j, carry):
        for q in range(8):
            sl = pl.ds(q * 16, 16)
            dn = plsc.load_gather(denv, [dst_v[j, sl]])
            w_v[j, sl] = w_v[j, sl] / (dn + 1e-16)
        return carry

    lax.fori_loop(0, _CPB, to_alpha, 0)

    _sc_half(t, src_v, dst_v, w_v, rows0, rows1, acc_sh,
             semg0, semg1, sems0, sems1, hsa_h, outa_h)
    _sc_half(t, src_v, dst_v, w_v, rows0, rows1, acc_sh,
             semg0, semg1, sems0, sems1, hsb_h, outb_h)


def _sc_scratch():
    return [
        pltpu.VMEM((_CPB, 128), jnp.int32),
        pltpu.VMEM((_CPB, 128), jnp.int32),
        pltpu.VMEM((_CPB, 128), jnp.float32),
        pltpu.VMEM((_NP,), jnp.float32),
        pltpu.VMEM((_NP,), jnp.float32),
        pltpu.VMEM((_NP,), jnp.float32),
        pltpu.VMEM((128,), jnp.float32),
        pltpu.VMEM((128, _S), jnp.float32),
        pltpu.VMEM((128, _S), jnp.float32),
        pltpu.VMEM_SHARED((_NP,), jnp.float32),
        pltpu.VMEM_SHARED((_NP, _S), jnp.float32),
        pltpu.SemaphoreType.DMA,
        pltpu.SemaphoreType.DMA,
        pltpu.SemaphoreType.DMA,
        pltpu.SemaphoreType.DMA,
    ]


def _sc_prologue(src_h, dst_h, t, src_v, dst_v, zrow_v):
    pltpu.sync_copy(src_h.at[t], src_v)
    pltpu.sync_copy(dst_h.at[t], dst_v)
    z16 = jnp.zeros((16,), jnp.float32)
    for q in range(8):
        zrow_v[pl.ds(q * 16, 16)] = z16


_SC_MESH = plsc.VectorSubcoreMesh(core_axis_name="c", subcore_axis_name="s")


@functools.partial(
    pl.kernel,
    out_type=jax.ShapeDtypeStruct((2 * _HEADS, _NP, _S), jnp.float32),
    mesh=_SC_MESH,
    compiler_params=pltpu.CompilerParams(
        needs_layout_passes=False, use_tc_tiling_on_sc=False),
    scratch_types=_sc_scratch(),
)
def _gat0(src_h, dst_h, as_h, ad_h, hs_h, out_h,
          src_v, dst_v, w_v, asv, adv, denv, zrow_v, rows0, rows1,
          den_sh, acc_sh, semg0, semg1, sems0, sems1):
    c = lax.axis_index("c")
    t = lax.axis_index("s")
    _sc_prologue(src_h, dst_h, t, src_v, dst_v, zrow_v)
    args = (src_v, dst_v, w_v, asv, adv, denv, zrow_v, rows0, rows1,
            den_sh, acc_sh, semg0, semg1, sems0, sems1)

    @pl.when(c == 0)
    def _():
        for g in (0, 1):
            _sc_group(t, *args, as_h.at[g], ad_h.at[g],
                      hs_h.at[2 * g], hs_h.at[2 * g + 1],
                      out_h.at[2 * g], out_h.at[2 * g + 1])

    @pl.when(c == 1)
    def _():
        for g in (2, 3):
            _sc_group(t, *args, as_h.at[g], ad_h.at[g],
                      hs_h.at[2 * g], hs_h.at[2 * g + 1],
                      out_h.at[2 * g], out_h.at[2 * g + 1])


@functools.partial(
    pl.kernel,
    out_type=jax.ShapeDtypeStruct((4, _NP, _S), jnp.float32),
    mesh=_SC_MESH,
    compiler_params=pltpu.CompilerParams(
        needs_layout_passes=False, use_tc_tiling_on_sc=False),
    scratch_types=_sc_scratch(),
)
def _gat1(src_h, dst_h, as_h, ad_h, h1_h, out_h,
          src_v, dst_v, w_v, asv, adv, denv, zrow_v, rows0, rows1,
          den_sh, acc_sh, semg0, semg1, sems0, sems1):
    c = lax.axis_index("c")
    t = lax.axis_index("s")
    _sc_prologue(src_h, dst_h, t, src_v, dst_v, zrow_v)
    args = (src_v, dst_v, w_v, asv, adv, denv, zrow_v, rows0, rows1,
            den_sh, acc_sh, semg0, semg1, sems0, sems1)

    @pl.when(c == 0)
    def _():
        _sc_group(t, *args, as_h.at[0], ad_h.at[0],
                  h1_h.at[0], h1_h.at[1], out_h.at[0], out_h.at[1])

    @pl.when(c == 1)
    def _():
        _sc_group(t, *args, as_h.at[0], ad_h.at[0],
                  h1_h.at[2], h1_h.at[3], out_h.at[2], out_h.at[3])


def kernel(x, edge_index, W_src0, W_dst0, att_src0, att_dst0, bias0,
           bn_gamma, bn_beta, bn_mean, bn_var, W1, att_src1, att_dst1,
           bias1):
    xp = jnp.pad(x, ((0, _NP - _N), (0, 0)))
    src = edge_index[0]
    dst = edge_index[1]
    srcp = jnp.concatenate(
        [src, jnp.zeros((_EP - _E,), jnp.int32)]).reshape(_TILES, _CPB, 128)
    dstp = jnp.concatenate(
        [dst, jnp.full((_EP - _E,), _PAD_DST, jnp.int32)]
    ).reshape(_TILES, _CPB, 128)

    hs, a_s, a_d = _proj0(xp, W_src0, W_dst0, att_src0, att_dst0)
    out0g = _gat0(srcp, dstp, a_s, a_d, hs)

    h1g, a1s, a1d = _mlp(
        out0g,
        bias0.reshape(2 * _HEADS, _S),
        bn_gamma.reshape(2 * _HEADS, _S),
        bn_beta.reshape(2 * _HEADS, _S),
        bn_mean.reshape(2 * _HEADS, _S),
        bn_var.reshape(2 * _HEADS, _S),
        W1.reshape(2 * _HEADS, _S, 2, _C),
        att_src1.reshape(2, _C),
        att_dst1.reshape(2, _C),
    )
    out1g = _gat1(srcp, dstp, a1s, a1d, h1g)
    out = out1g.transpose(1, 0, 2).reshape(_NP, 4 * _S)[:_N] + bias1
    return out
